# Initial kernel scaffold; baseline (speedup 1.0000x reference)
import jax
import jax.numpy as jnp
from jax.experimental import pallas as pl


def kernel(x, edge_index, batch, W_l, b_l, W_r, W_lin, b_lin):
    raise NotImplementedError("write your pallas kernel here")


# speedup vs baseline: 4.7851x; 4.7851x over previous
import functools

import jax
import jax.numpy as jnp
from jax import lax
from jax.experimental import pallas as pl
from jax.experimental.pallas import tpu as pltpu
from jax.experimental.pallas import tpu_sc as plsc

N = 10000
E = 160000
D = 256
H = 512
C = 10
G = 128

NC = 2
NS = 16
L = 16

NPAD = 10240
EPT = E // NS
EPTP = 10240
ECH = EPTP // 128
DH = D // 2
GP = 144

_mesh = plsc.VectorSubcoreMesh(core_axis_name="c", subcore_axis_name="s")


def _agg_kernel(x2, srcp, dstp, z2d, z1d, agg2, degp,
                src_v, dst_v, rows_v, ones_v, agg_sh, deg_sh):
    c = lax.axis_index("c")
    s = lax.axis_index("s")
    ones16 = jnp.full((L,), 1.0, jnp.float32)

    pltpu.sync_copy(z2d, agg_sh.at[pl.ds(s * 640, 640)])
    pltpu.sync_copy(srcp.at[s], src_v)
    pltpu.sync_copy(dstp.at[s], dst_v)
    for t in range(128 // L):
        ones_v[pl.ds(t * L, L)] = ones16

    @pl.when(c == 0)
    def _():
        pltpu.sync_copy(z1d.at[pl.ds(s * 640, 640)],
                        deg_sh.at[pl.ds(s * 640, 640)])

    plsc.subcore_barrier()

    def chunk(k, carry):
        pltpu.sync_copy(x2.at[c].at[src_v.at[k]], rows_v)
        pltpu.sync_copy(rows_v, agg_sh.at[dst_v.at[k]], add=True)

        @pl.when(c == 0)
        def _():
            pltpu.sync_copy(ones_v, deg_sh.at[dst_v.at[k]], add=True)

        return carry

    lax.fori_loop(0, ECH, chunk, 0, unroll=False)

    plsc.subcore_barrier()

    pltpu.sync_copy(agg_sh.at[pl.ds(s * 640, 640)],
                    agg2.at[c].at[pl.ds(s * 640, 640)])

    @pl.when(c == 0)
    def _():
        pltpu.sync_copy(deg_sh.at[pl.ds(s * 640, 640)],
                        degp.at[pl.ds(s * 640, 640)])


_agg_call = pl.kernel(
    _agg_kernel,
    out_type=[
        jax.ShapeDtypeStruct((NC, NPAD, DH), jnp.float32),
        jax.ShapeDtypeStruct((NPAD,), jnp.float32),
    ],
    mesh=_mesh,
    scratch_types=[
        pltpu.VMEM((ECH, 128), jnp.int32),
        pltpu.VMEM((ECH, 128), jnp.int32),
        pltpu.VMEM((128, DH), jnp.float32),
        pltpu.VMEM((128,), jnp.float32),
        pltpu.VMEM_SHARED((NPAD, DH), jnp.float32),
        pltpu.VMEM_SHARED((NPAD,), jnp.float32),
    ],
    compiler_params=pltpu.CompilerParams(needs_layout_passes=False),
)


def _mm_kernel(agg_ref, x_ref, deg_ref, wl_ref, wr_ref, bl_ref, out_ref):
    inv = 1.0 / jnp.maximum(deg_ref[...], 1.0)
    ml = agg_ref[0] * inv
    mh = agg_ref[1] * inv
    acc = jnp.dot(ml, wl_ref[:DH], preferred_element_type=jnp.float32)
    acc += jnp.dot(mh, wl_ref[DH:], preferred_element_type=jnp.float32)
    acc += jnp.dot(x_ref[...], wr_ref[...], preferred_element_type=jnp.float32)
    acc += bl_ref[...]
    out_ref[...] = acc * 0.5 * (1.0 + lax.erf(acc * (2.0 ** -0.5)))


def _pool_kernel(h, batchp, pooled,
                 b_v, cnt_loc, cnt_dma, cntall_v, hbuf, acc_ts, pool_buf,
                 cntall_sh, pooled_sh):
    c = lax.axis_index("c")
    s = lax.axis_index("s")
    w = c * NS + s
    ones16 = jnp.full((L,), 1.0, jnp.float32)
    iota16 = lax.broadcasted_iota(jnp.int32, (L,), 0)

    pltpu.sync_copy(batchp.at[s], b_v)
    for t in range(GP // L):
        cnt_loc[pl.ds(t * L, L)] = jnp.zeros((L,), jnp.float32)
    for j in range(640 // L):
        idx16 = b_v[pl.ds(j * L, L)]
        plsc.addupdate_scatter(cnt_loc, [idx16], ones16)
    for t in range(GP // L):
        cnt_dma[pl.ds(t * L, L)] = cnt_loc[pl.ds(t * L, L)]
    pltpu.sync_copy(cnt_dma, cntall_sh.at[s])
    plsc.subcore_barrier()
    pltpu.sync_copy(cntall_sh, cntall_v)

    cnt_chunks = []
    for t in range(GP // L):
        acc = cntall_v[0, pl.ds(t * L, L)]
        for p in range(1, NS):
            acc = acc + cntall_v[p, pl.ds(t * L, L)]
        cnt_chunks.append(acc)

    def graph_scalars(g):
        start = jnp.float32(0)
        cnt = jnp.float32(0)
        for t, ch in enumerate(cnt_chunks):
            ids = t * L + iota16
            start = start + jnp.sum(jnp.where(ids < g, ch, 0.0), axis=0)
            cnt = cnt + jnp.sum(jnp.where(ids == g, ch, 0.0), axis=0)
        return start.astype(jnp.int32), cnt.astype(jnp.int32), cnt

    zero16 = jnp.zeros((L,), jnp.float32)
    pinf16 = jnp.full((L,), jnp.inf, jnp.float32)
    ninf16 = jnp.full((L,), -jnp.inf, jnp.float32)

    for i in range(4):
        g = w * 4 + i
        start, cnt, cnt_f = graph_scalars(g)

        for t in range(H // L):
            acc_ts[0, pl.ds(t * L, L)] = zero16
            acc_ts[1, pl.ds(t * L, L)] = pinf16
            acc_ts[2, pl.ds(t * L, L)] = ninf16

        def body(state):
            p = state
            base = start + p
            a = (base // 8) * 8
            off = base - a
            pltpu.sync_copy(h.at[pl.ds(a, 72)], hbuf)
            rows_here = jnp.minimum(64, cnt - p)
            for cg in range(4):
                accs = []
                for k in range(8):
                    col = cg * 128 + k * L
                    accs.append(acc_ts[0, pl.ds(col, L)])
                    accs.append(acc_ts[1, pl.ds(col, L)])
                    accs.append(acc_ts[2, pl.ds(col, L)])

                def row_body(r, accs):
                    accs = list(accs)
                    for k in range(8):
                        col = cg * 128 + k * L
                        v = hbuf[off + r, pl.ds(col, L)]
                        accs[3 * k] = accs[3 * k] + v
                        accs[3 * k + 1] = jnp.minimum(accs[3 * k + 1], v)
                        accs[3 * k + 2] = jnp.maximum(accs[3 * k + 2], v)
                    return tuple(accs)

                accs = lax.fori_loop(0, rows_here, row_body, tuple(accs),
                                     unroll=False)
                for k in range(8):
                    col = cg * 128 + k * L
                    acc_ts[0, pl.ds(col, L)] = accs[3 * k]
                    acc_ts[1, pl.ds(col, L)] = accs[3 * k + 1]
                    acc_ts[2, pl.ds(col, L)] = accs[3 * k + 2]
            return p + 64

        lax.while_loop(lambda p: p < cnt, body, jnp.int32(0))

        cnt16 = jnp.zeros((L,), jnp.float32) + jnp.maximum(cnt_f, 1.0)
        for t in range(H // L):
            pool_buf[i, pl.ds(t * L, L)] = acc_ts[0, pl.ds(t * L, L)] / cnt16
            pool_buf[i, pl.ds(H + t * L, L)] = acc_ts[1, pl.ds(t * L, L)]
            pool_buf[i, pl.ds(2 * H + t * L, L)] = acc_ts[2, pl.ds(t * L, L)]

    pltpu.sync_copy(pool_buf, pooled_sh.at[pl.ds(s * 4, 4)])
    plsc.subcore_barrier()

    @pl.when(s < 8)
    def _():
        pltpu.sync_copy(pooled_sh.at[pl.ds(s * 8, 8)],
                        pooled.at[pl.ds(c * 64 + s * 8, 8)])


_pool_call = pl.kernel(
    _pool_kernel,
    out_type=[jax.ShapeDtypeStruct((G, 3 * H), jnp.float32)],
    mesh=_mesh,
    scratch_types=[
        pltpu.VMEM((640,), jnp.int32),
        pltpu.VMEM((GP,), jnp.float32),
        pltpu.VMEM((GP,), jnp.float32),
        pltpu.VMEM((NS, GP), jnp.float32),
        pltpu.VMEM((72, H), jnp.float32),
        pltpu.VMEM((3, H), jnp.float32),
        pltpu.VMEM((4, 3 * H), jnp.float32),
        pltpu.VMEM_SHARED((NS, GP), jnp.float32),
        pltpu.VMEM_SHARED((64, 3 * H), jnp.float32),
    ],
    compiler_params=pltpu.CompilerParams(needs_layout_passes=False),
)


def _fin_kernel(p_ref, w_ref, b_ref, out_ref):
    out_ref[...] = jnp.dot(p_ref[...], w_ref[...],
                           preferred_element_type=jnp.float32) + b_ref[...]


def kernel(x, edge_index, batch, W_l, b_l, W_r, W_lin, b_lin):
    src = edge_index[0]
    dst = edge_index[1]

    x_pad = jnp.pad(x, ((0, NPAD - N), (0, 0)))
    x2 = x_pad.reshape(NPAD, 2, DH).transpose(1, 0, 2)
    srcp = jnp.pad(src.reshape(NS, EPT), ((0, 0), (0, EPTP - EPT))
                   ).reshape(NS, ECH, 128)
    dstp = jnp.pad(dst.reshape(NS, EPT), ((0, 0), (0, EPTP - EPT)),
                   constant_values=NPAD - 8).reshape(NS, ECH, 128)
    z2d = jnp.zeros((640, DH), jnp.float32)
    z1d = jnp.zeros((NPAD,), jnp.float32)

    agg2, degp = _agg_call(x2, srcp, dstp, z2d, z1d)

    degp2 = degp.reshape(NPAD, 1)
    bl2 = b_l.reshape(1, H)
    h = pl.pallas_call(
        _mm_kernel,
        grid=(NPAD // 256,),
        in_specs=[
            pl.BlockSpec((NC, 256, DH), lambda i: (0, i, 0)),
            pl.BlockSpec((256, D), lambda i: (i, 0)),
            pl.BlockSpec((256, 1), lambda i: (i, 0)),
            pl.BlockSpec((D, H), lambda i: (0, 0)),
            pl.BlockSpec((D, H), lambda i: (0, 0)),
            pl.BlockSpec((1, H), lambda i: (0, 0)),
        ],
        out_specs=pl.BlockSpec((256, H), lambda i: (i, 0)),
        out_shape=jax.ShapeDtypeStruct((NPAD, H), jnp.float32),
    )(agg2, x_pad, degp2, W_l, W_r, bl2)

    batchp = jnp.pad(batch, (0, NPAD - N), constant_values=G).reshape(NS, 640)
    (pooled,) = _pool_call(h, batchp)

    CP = 128
    w_cat = jnp.pad(W_lin, ((0, 0), (0, CP - C)))
    b_cat = jnp.pad(b_lin, (0, CP - C)).reshape(1, CP)
    out = pl.pallas_call(
        _fin_kernel,
        out_shape=jax.ShapeDtypeStruct((G, CP), jnp.float32),
    )(pooled, w_cat, b_cat)
    return out[:, :C]


# speedup vs baseline: 5.2756x; 1.1025x over previous
import functools

import jax
import jax.numpy as jnp
from jax import lax
from jax.experimental import pallas as pl
from jax.experimental.pallas import tpu as pltpu
from jax.experimental.pallas import tpu_sc as plsc

N = 10000
E = 160000
D = 256
H = 512
C = 10
G = 128

NC = 2
NS = 16
L = 16

NPAD = 10240
EPT = E // NS
EPTP = 10240
CH = 64
ECH = EPTP // CH
DH = D // 2
GP = 144
NPA = 10112

_mesh = plsc.VectorSubcoreMesh(core_axis_name="c", subcore_axis_name="s")


def _agg_kernel(x2, srcp, dstp, z2d, z1d, agg2, degp,
                src_v, dst_v, rows0, rows1, ones_v, gsem0, gsem1,
                agg_sh, deg_sh):
    c = lax.axis_index("c")
    s = lax.axis_index("s")
    ones16 = jnp.full((L,), 1.0, jnp.float32)

    pltpu.sync_copy(z2d.at[pl.ds(0, 632)], agg_sh.at[pl.ds(s * 632, 632)])
    for t in range(CH // L):
        ones_v[pl.ds(t * L, L)] = ones16

    @pl.when(c == 0)
    def _():
        pltpu.sync_copy(z1d.at[pl.ds(s * 640, 640)],
                        deg_sh.at[pl.ds(s * 640, 640)])

    plsc.subcore_barrier()

    rows = (rows0, rows1)
    gsems = (gsem0, gsem1)
    HC = ECH // 2

    for half in range(2):
        pltpu.sync_copy(srcp.at[s].at[half], src_v)
        pltpu.sync_copy(dstp.at[s].at[half], dst_v)
        pltpu.async_copy(x2.at[c].at[src_v.at[0]], rows0, gsem0)

        def super_chunk(g, carry):
            for b in range(2):
                k = g * 2 + b
                @pl.when(k + 1 < HC)
                def _():
                    pltpu.async_copy(x2.at[c].at[src_v.at[k + 1]],
                                     rows[1 - b], gsems[1 - b])

                pltpu.make_async_copy(x2.at[c].at[src_v.at[k]],
                                      rows[b], gsems[b]).wait()
                pltpu.sync_copy(rows[b], agg_sh.at[dst_v.at[k]], add=True)

                @pl.when(c == 0)
                def _():
                    pltpu.sync_copy(ones_v, deg_sh.at[dst_v.at[k]], add=True)

            return carry

        lax.fori_loop(0, HC // 2, super_chunk, 0, unroll=False)

    plsc.subcore_barrier()

    pltpu.sync_copy(agg_sh.at[pl.ds(s * 632, 632)],
                    agg2.at[c].at[pl.ds(s * 632, 632)])

    @pl.when(c == 0)
    def _():
        pltpu.sync_copy(deg_sh.at[pl.ds(s * 640, 640)],
                        degp.at[pl.ds(s * 640, 640)])


_agg_call = pl.kernel(
    _agg_kernel,
    out_type=[
        jax.ShapeDtypeStruct((NC, NPAD, DH), jnp.float32),
        jax.ShapeDtypeStruct((NPAD,), jnp.float32),
    ],
    mesh=_mesh,
    scratch_types=[
        pltpu.VMEM((ECH // 2, CH), jnp.int32),
        pltpu.VMEM((ECH // 2, CH), jnp.int32),
        pltpu.VMEM((CH, DH), jnp.float32),
        pltpu.VMEM((CH, DH), jnp.float32),
        pltpu.VMEM((CH,), jnp.float32),
        pltpu.SemaphoreType.DMA,
        pltpu.SemaphoreType.DMA,
        pltpu.VMEM_SHARED((NPA, DH), jnp.float32),
        pltpu.VMEM_SHARED((NPAD,), jnp.float32),
    ],
    compiler_params=pltpu.CompilerParams(needs_layout_passes=False),
)


def _mm_kernel(agg_ref, x_ref, deg_ref, wl_ref, wr_ref, bl_ref, out_ref):
    inv = 1.0 / jnp.maximum(deg_ref[...], 1.0)
    ml = agg_ref[0] * inv
    mh = agg_ref[1] * inv
    acc = jnp.dot(ml, wl_ref[:DH], preferred_element_type=jnp.float32)
    acc += jnp.dot(mh, wl_ref[DH:], preferred_element_type=jnp.float32)
    acc += jnp.dot(x_ref[...], wr_ref[...], preferred_element_type=jnp.float32)
    acc += bl_ref[...]
    out_ref[...] = acc * 0.5 * (1.0 + lax.erf(acc * (2.0 ** -0.5)))


def _pool_kernel(h, batchp, pooled,
                 b_v, cnt_loc, cnt_dma, cntall_v, hbuf, acc_ts, pool_buf,
                 cntall_sh, pooled_sh):
    c = lax.axis_index("c")
    s = lax.axis_index("s")
    w = c * NS + s
    ones16 = jnp.full((L,), 1.0, jnp.float32)
    iota16 = lax.broadcasted_iota(jnp.int32, (L,), 0)

    pltpu.sync_copy(batchp.at[s], b_v)
    for t in range(GP // L):
        cnt_loc[pl.ds(t * L, L)] = jnp.zeros((L,), jnp.float32)
    for j in range(640 // L):
        idx16 = b_v[pl.ds(j * L, L)]
        plsc.addupdate_scatter(cnt_loc, [idx16], ones16)
    for t in range(GP // L):
        cnt_dma[pl.ds(t * L, L)] = cnt_loc[pl.ds(t * L, L)]
    pltpu.sync_copy(cnt_dma, cntall_sh.at[s])
    plsc.subcore_barrier()
    pltpu.sync_copy(cntall_sh, cntall_v)

    cnt_chunks = []
    for t in range(GP // L):
        acc = cntall_v[0, pl.ds(t * L, L)]
        for p in range(1, NS):
            acc = acc + cntall_v[p, pl.ds(t * L, L)]
        cnt_chunks.append(acc)

    def graph_scalars(g):
        start = jnp.float32(0)
        cnt = jnp.float32(0)
        for t, ch in enumerate(cnt_chunks):
            ids = t * L + iota16
            start = start + jnp.sum(jnp.where(ids < g, ch, 0.0), axis=0)
            cnt = cnt + jnp.sum(jnp.where(ids == g, ch, 0.0), axis=0)
        return start.astype(jnp.int32), cnt.astype(jnp.int32), cnt

    zero16 = jnp.zeros((L,), jnp.float32)
    pinf16 = jnp.full((L,), jnp.inf, jnp.float32)
    ninf16 = jnp.full((L,), -jnp.inf, jnp.float32)

    for i in range(4):
        g = w * 4 + i
        start, cnt, cnt_f = graph_scalars(g)

        for t in range(H // L):
            acc_ts[0, pl.ds(t * L, L)] = zero16
            acc_ts[1, pl.ds(t * L, L)] = pinf16
            acc_ts[2, pl.ds(t * L, L)] = ninf16

        def body(state):
            p = state
            base = start + p
            a = (base // 8) * 8
            off = base - a
            pltpu.sync_copy(h.at[pl.ds(a, 72)], hbuf)
            rows_here = jnp.minimum(64, cnt - p)
            for cg in range(4):
                accs = []
                for k in range(8):
                    col = cg * 128 + k * L
                    accs.append(acc_ts[0, pl.ds(col, L)])
                    accs.append(acc_ts[1, pl.ds(col, L)])
                    accs.append(acc_ts[2, pl.ds(col, L)])

                def row_body(r, accs):
                    accs = list(accs)
                    for k in range(8):
                        col = cg * 128 + k * L
                        v = hbuf[off + r, pl.ds(col, L)]
                        accs[3 * k] = accs[3 * k] + v
                        accs[3 * k + 1] = jnp.minimum(accs[3 * k + 1], v)
                        accs[3 * k + 2] = jnp.maximum(accs[3 * k + 2], v)
                    return tuple(accs)

                accs = lax.fori_loop(0, rows_here, row_body, tuple(accs),
                                     unroll=False)
                for k in range(8):
                    col = cg * 128 + k * L
                    acc_ts[0, pl.ds(col, L)] = accs[3 * k]
                    acc_ts[1, pl.ds(col, L)] = accs[3 * k + 1]
                    acc_ts[2, pl.ds(col, L)] = accs[3 * k + 2]
            return p + 64

        lax.while_loop(lambda p: p < cnt, body, jnp.int32(0))

        cnt16 = jnp.zeros((L,), jnp.float32) + jnp.maximum(cnt_f, 1.0)
        for t in range(H // L):
            pool_buf[i, pl.ds(t * L, L)] = acc_ts[0, pl.ds(t * L, L)] / cnt16
            pool_buf[i, pl.ds(H + t * L, L)] = acc_ts[1, pl.ds(t * L, L)]
            pool_buf[i, pl.ds(2 * H + t * L, L)] = acc_ts[2, pl.ds(t * L, L)]

    pltpu.sync_copy(pool_buf, pooled_sh.at[pl.ds(s * 4, 4)])
    plsc.subcore_barrier()

    @pl.when(s < 8)
    def _():
        pltpu.sync_copy(pooled_sh.at[pl.ds(s * 8, 8)],
                        pooled.at[pl.ds(c * 64 + s * 8, 8)])


_pool_call = pl.kernel(
    _pool_kernel,
    out_type=[jax.ShapeDtypeStruct((G, 3 * H), jnp.float32)],
    mesh=_mesh,
    scratch_types=[
        pltpu.VMEM((640,), jnp.int32),
        pltpu.VMEM((GP,), jnp.float32),
        pltpu.VMEM((GP,), jnp.float32),
        pltpu.VMEM((NS, GP), jnp.float32),
        pltpu.VMEM((72, H), jnp.float32),
        pltpu.VMEM((3, H), jnp.float32),
        pltpu.VMEM((4, 3 * H), jnp.float32),
        pltpu.VMEM_SHARED((NS, GP), jnp.float32),
        pltpu.VMEM_SHARED((64, 3 * H), jnp.float32),
    ],
    compiler_params=pltpu.CompilerParams(needs_layout_passes=False),
)


def _fin_kernel(p_ref, w_ref, b_ref, out_ref):
    out_ref[...] = jnp.dot(p_ref[...], w_ref[...],
                           preferred_element_type=jnp.float32) + b_ref[...]


def kernel(x, edge_index, batch, W_l, b_l, W_r, W_lin, b_lin):
    src = edge_index[0]
    dst = edge_index[1]

    x_pad = jnp.pad(x, ((0, NPAD - N), (0, 0)))
    x2 = x_pad.reshape(NPAD, 2, DH).transpose(1, 0, 2)
    srcp = jnp.pad(src.reshape(NS, EPT), ((0, 0), (0, EPTP - EPT))
                   ).reshape(NS, 2, ECH // 2, CH)
    dstp = jnp.pad(dst.reshape(NS, EPT), ((0, 0), (0, EPTP - EPT)),
                   constant_values=NPA - 8).reshape(NS, 2, ECH // 2, CH)
    z2d = jnp.zeros((640, DH), jnp.float32)
    z1d = jnp.zeros((NPAD,), jnp.float32)

    agg2, degp = _agg_call(x2, srcp, dstp, z2d, z1d)

    degp2 = degp.reshape(NPAD, 1)
    bl2 = b_l.reshape(1, H)
    h = pl.pallas_call(
        _mm_kernel,
        grid=(NPAD // 256,),
        in_specs=[
            pl.BlockSpec((NC, 256, DH), lambda i: (0, i, 0)),
            pl.BlockSpec((256, D), lambda i: (i, 0)),
            pl.BlockSpec((256, 1), lambda i: (i, 0)),
            pl.BlockSpec((D, H), lambda i: (0, 0)),
            pl.BlockSpec((D, H), lambda i: (0, 0)),
            pl.BlockSpec((1, H), lambda i: (0, 0)),
        ],
        out_specs=pl.BlockSpec((256, H), lambda i: (i, 0)),
        out_shape=jax.ShapeDtypeStruct((NPAD, H), jnp.float32),
    )(agg2, x_pad, degp2, W_l, W_r, bl2)

    batchp = jnp.pad(batch, (0, NPAD - N), constant_values=G).reshape(NS, 640)
    (pooled,) = _pool_call(h, batchp)

    CP = 128
    w_cat = jnp.pad(W_lin, ((0, 0), (0, CP - C)))
    b_cat = jnp.pad(b_lin, (0, CP - C)).reshape(1, CP)
    out = pl.pallas_call(
        _fin_kernel,
        out_shape=jax.ShapeDtypeStruct((G, CP), jnp.float32),
    )(pooled, w_cat, b_cat)
    return out[:, :C]


# speedup vs baseline: 5.4735x; 1.0375x over previous
import functools

import jax
import jax.numpy as jnp
from jax import lax
from jax.experimental import pallas as pl
from jax.experimental.pallas import tpu as pltpu
from jax.experimental.pallas import tpu_sc as plsc

N = 10000
E = 160000
D = 256
H = 512
C = 10
G = 128

NC = 2
NS = 16
L = 16

NPAD = 10240
EPT = E // NS
EPTP = 10240
CH = 80
ECH = EPTP // CH
NQ = 4
CPQ = ECH // NQ
DH = D // 2
GP = 144
NPA = 10112

_mesh = plsc.VectorSubcoreMesh(core_axis_name="c", subcore_axis_name="s")


def _agg_kernel(x2, srcp, dstp, z2d, z1d, agg2, degp,
                src_v, dst_v, rows0, rows1, ones_v,
                gsem0, gsem1, ssem0, ssem1, dsem0, dsem1,
                agg_sh, deg_sh):
    c = lax.axis_index("c")
    s = lax.axis_index("s")
    ones16 = jnp.full((L,), 1.0, jnp.float32)

    pltpu.sync_copy(z2d.at[pl.ds(0, 632)], agg_sh.at[pl.ds(s * 632, 632)])
    pltpu.sync_copy(z1d.at[pl.ds(s * 640, 640)],
                    deg_sh.at[pl.ds(s * 640, 640)])
    for t in range(CH // L):
        ones_v[pl.ds(t * L, L)] = ones16

    plsc.subcore_barrier()

    rows = (rows0, rows1)
    gsems = (gsem0, gsem1)
    ssems = (ssem0, ssem1)
    dsems = (dsem0, dsem1)

    def gather_start(k, b):
        pltpu.async_copy(x2.at[c].at[src_v.at[k]], rows[b], gsems[b])

    def gather_wait(b):
        pltpu.make_async_copy(x2.at[c].at[src_v.at[0]], rows[b],
                              gsems[b]).wait()

    def scatter_start(k, b):
        pltpu.async_copy(rows[b], agg_sh.at[dst_v.at[k]], ssems[b], add=True)

    def scatter_wait(b):
        pltpu.make_async_copy(rows[b], agg_sh.at[dst_v.at[0]],
                              ssems[b]).wait()

    def degsc_start(k, b):
        pltpu.async_copy(ones_v, deg_sh.at[dst_v.at[k]], dsems[b], add=True)

    def degsc_wait(b):
        pltpu.make_async_copy(ones_v, deg_sh.at[dst_v.at[0]],
                              dsems[b]).wait()

    for q in range(NQ):
        pltpu.sync_copy(srcp.at[s].at[q], src_v)
        pltpu.sync_copy(dstp.at[s].at[q], dst_v)
        deg_core = 0 if q < NQ // 2 else 1
        pltpu.async_copy(x2.at[c].at[src_v.at[0]], rows0, gsem0)

        def super_chunk(g, carry):
            for b in range(2):
                k = g * 2 + b

                @pl.when(k >= 1)
                def _():
                    scatter_wait(1 - b)

                @pl.when(k + 1 < CPQ)
                def _():
                    gather_start(k + 1, 1 - b)

                gather_wait(b)
                scatter_start(k, b)

                @pl.when(c == deg_core)
                def _():
                    @pl.when(k >= 1)
                    def _():
                        degsc_wait(1 - b)
                    degsc_start(k, b)

            return carry

        lax.fori_loop(0, CPQ // 2, super_chunk, 0, unroll=False)

        scatter_wait(1)

        @pl.when(c == deg_core)
        def _():
            degsc_wait(1)

    plsc.subcore_barrier()

    pltpu.sync_copy(agg_sh.at[pl.ds(s * 632, 632)],
                    agg2.at[c].at[pl.ds(s * 632, 632)])
    pltpu.sync_copy(deg_sh.at[pl.ds(s * 640, 640)],
                    degp.at[c].at[pl.ds(s * 640, 640)])


_agg_call = pl.kernel(
    _agg_kernel,
    out_type=[
        jax.ShapeDtypeStruct((NC, NPAD, DH), jnp.float32),
        jax.ShapeDtypeStruct((NC, NPAD), jnp.float32),
    ],
    mesh=_mesh,
    scratch_types=[
        pltpu.VMEM((CPQ, CH), jnp.int32),
        pltpu.VMEM((CPQ, CH), jnp.int32),
        pltpu.VMEM((CH, DH), jnp.float32),
        pltpu.VMEM((CH, DH), jnp.float32),
        pltpu.VMEM((CH,), jnp.float32),
        pltpu.SemaphoreType.DMA,
        pltpu.SemaphoreType.DMA,
        pltpu.SemaphoreType.DMA,
        pltpu.SemaphoreType.DMA,
        pltpu.SemaphoreType.DMA,
        pltpu.SemaphoreType.DMA,
        pltpu.VMEM_SHARED((NPA, DH), jnp.float32),
        pltpu.VMEM_SHARED((NPAD,), jnp.float32),
    ],
    compiler_params=pltpu.CompilerParams(needs_layout_passes=False),
)


def _mm_kernel(agg_ref, x_ref, deg_ref, wl_ref, wr_ref, bl_ref, out_ref):
    deg = deg_ref[0] + deg_ref[1]
    inv = 1.0 / jnp.maximum(deg, 1.0)
    ml = agg_ref[0] * inv
    mh = agg_ref[1] * inv
    acc = jnp.dot(ml, wl_ref[:DH], preferred_element_type=jnp.float32)
    acc += jnp.dot(mh, wl_ref[DH:], preferred_element_type=jnp.float32)
    acc += jnp.dot(x_ref[...], wr_ref[...], preferred_element_type=jnp.float32)
    acc += bl_ref[...]
    out_ref[...] = acc * 0.5 * (1.0 + lax.erf(acc * (2.0 ** -0.5)))


def _pool_kernel(h, batchp, pooled,
                 b_v, cnt_loc, cnt_dma, cntall_v, hbuf, acc_ts, pool_buf,
                 cntall_sh, pooled_sh):
    c = lax.axis_index("c")
    s = lax.axis_index("s")
    w = c * NS + s
    ones16 = jnp.full((L,), 1.0, jnp.float32)
    iota16 = lax.broadcasted_iota(jnp.int32, (L,), 0)

    pltpu.sync_copy(batchp.at[s], b_v)
    for t in range(GP // L):
        cnt_loc[pl.ds(t * L, L)] = jnp.zeros((L,), jnp.float32)
    for j in range(640 // L):
        idx16 = b_v[pl.ds(j * L, L)]
        plsc.addupdate_scatter(cnt_loc, [idx16], ones16)
    for t in range(GP // L):
        cnt_dma[pl.ds(t * L, L)] = cnt_loc[pl.ds(t * L, L)]
    pltpu.sync_copy(cnt_dma, cntall_sh.at[s])
    plsc.subcore_barrier()
    pltpu.sync_copy(cntall_sh, cntall_v)

    cnt_chunks = []
    for t in range(GP // L):
        acc = cntall_v[0, pl.ds(t * L, L)]
        for p in range(1, NS):
            acc = acc + cntall_v[p, pl.ds(t * L, L)]
        cnt_chunks.append(acc)

    def graph_scalars(g):
        start = jnp.float32(0)
        cnt = jnp.float32(0)
        for t, ch in enumerate(cnt_chunks):
            ids = t * L + iota16
            start = start + jnp.sum(jnp.where(ids < g, ch, 0.0), axis=0)
            cnt = cnt + jnp.sum(jnp.where(ids == g, ch, 0.0), axis=0)
        return start.astype(jnp.int32), cnt.astype(jnp.int32), cnt

    zero16 = jnp.zeros((L,), jnp.float32)
    pinf16 = jnp.full((L,), jnp.inf, jnp.float32)
    ninf16 = jnp.full((L,), -jnp.inf, jnp.float32)

    for i in range(4):
        g = w * 4 + i
        start, cnt, cnt_f = graph_scalars(g)

        for t in range(H // L):
            acc_ts[0, pl.ds(t * L, L)] = zero16
            acc_ts[1, pl.ds(t * L, L)] = pinf16
            acc_ts[2, pl.ds(t * L, L)] = ninf16

        def body(state):
            p = state
            base = start + p
            a = (base // 8) * 8
            off = base - a
            pltpu.sync_copy(h.at[pl.ds(a, 72)], hbuf)
            rows_here = jnp.minimum(64, cnt - p)
            for cg in range(4):
                accs = []
                for k in range(8):
                    col = cg * 128 + k * L
                    accs.append(acc_ts[0, pl.ds(col, L)])
                    accs.append(acc_ts[1, pl.ds(col, L)])
                    accs.append(acc_ts[2, pl.ds(col, L)])

                def row_body(r, accs):
                    accs = list(accs)
                    for k in range(8):
                        col = cg * 128 + k * L
                        v = hbuf[off + r, pl.ds(col, L)]
                        accs[3 * k] = accs[3 * k] + v
                        accs[3 * k + 1] = jnp.minimum(accs[3 * k + 1], v)
                        accs[3 * k + 2] = jnp.maximum(accs[3 * k + 2], v)
                    return tuple(accs)

                accs = lax.fori_loop(0, rows_here, row_body, tuple(accs),
                                     unroll=False)
                for k in range(8):
                    col = cg * 128 + k * L
                    acc_ts[0, pl.ds(col, L)] = accs[3 * k]
                    acc_ts[1, pl.ds(col, L)] = accs[3 * k + 1]
                    acc_ts[2, pl.ds(col, L)] = accs[3 * k + 2]
            return p + 64

        lax.while_loop(lambda p: p < cnt, body, jnp.int32(0))

        cnt16 = jnp.zeros((L,), jnp.float32) + jnp.maximum(cnt_f, 1.0)
        for t in range(H // L):
            pool_buf[i, pl.ds(t * L, L)] = acc_ts[0, pl.ds(t * L, L)] / cnt16
            pool_buf[i, pl.ds(H + t * L, L)] = acc_ts[1, pl.ds(t * L, L)]
            pool_buf[i, pl.ds(2 * H + t * L, L)] = acc_ts[2, pl.ds(t * L, L)]

    pltpu.sync_copy(pool_buf, pooled_sh.at[pl.ds(s * 4, 4)])
    plsc.subcore_barrier()

    @pl.when(s < 8)
    def _():
        pltpu.sync_copy(pooled_sh.at[pl.ds(s * 8, 8)],
                        pooled.at[pl.ds(c * 64 + s * 8, 8)])


_pool_call = pl.kernel(
    _pool_kernel,
    out_type=[jax.ShapeDtypeStruct((G, 3 * H), jnp.float32)],
    mesh=_mesh,
    scratch_types=[
        pltpu.VMEM((640,), jnp.int32),
        pltpu.VMEM((GP,), jnp.float32),
        pltpu.VMEM((GP,), jnp.float32),
        pltpu.VMEM((NS, GP), jnp.float32),
        pltpu.VMEM((72, H), jnp.float32),
        pltpu.VMEM((3, H), jnp.float32),
        pltpu.VMEM((4, 3 * H), jnp.float32),
        pltpu.VMEM_SHARED((NS, GP), jnp.float32),
        pltpu.VMEM_SHARED((64, 3 * H), jnp.float32),
    ],
    compiler_params=pltpu.CompilerParams(needs_layout_passes=False),
)


def _fin_kernel(p_ref, w_ref, b_ref, out_ref):
    out_ref[...] = jnp.dot(p_ref[...], w_ref[...],
                           preferred_element_type=jnp.float32) + b_ref[...]


def kernel(x, edge_index, batch, W_l, b_l, W_r, W_lin, b_lin):
    src = edge_index[0]
    dst = edge_index[1]

    x_pad = jnp.pad(x, ((0, NPAD - N), (0, 0)))
    x2 = x_pad.reshape(NPAD, 2, DH).transpose(1, 0, 2)
    srcp = jnp.pad(src.reshape(NS, EPT), ((0, 0), (0, EPTP - EPT))
                   ).reshape(NS, NQ, CPQ, CH)
    dstp = jnp.pad(dst.reshape(NS, EPT), ((0, 0), (0, EPTP - EPT)),
                   constant_values=NPA - 8).reshape(NS, NQ, CPQ, CH)
    z2d = jnp.zeros((640, DH), jnp.float32)
    z1d = jnp.zeros((NPAD,), jnp.float32)

    agg2, degp = _agg_call(x2, srcp, dstp, z2d, z1d)

    degp2 = degp.reshape(NC, NPAD, 1)
    bl2 = b_l.reshape(1, H)
    h = pl.pallas_call(
        _mm_kernel,
        grid=(NPAD // 256,),
        in_specs=[
            pl.BlockSpec((NC, 256, DH), lambda i: (0, i, 0)),
            pl.BlockSpec((256, D), lambda i: (i, 0)),
            pl.BlockSpec((NC, 256, 1), lambda i: (0, i, 0)),
            pl.BlockSpec((D, H), lambda i: (0, 0)),
            pl.BlockSpec((D, H), lambda i: (0, 0)),
            pl.BlockSpec((1, H), lambda i: (0, 0)),
        ],
        out_specs=pl.BlockSpec((256, H), lambda i: (i, 0)),
        out_shape=jax.ShapeDtypeStruct((NPAD, H), jnp.float32),
    )(agg2, x_pad, degp2, W_l, W_r, bl2)

    batchp = jnp.pad(batch, (0, NPAD - N), constant_values=G).reshape(NS, 640)
    (pooled,) = _pool_call(h, batchp)

    CP = 128
    w_cat = jnp.pad(W_lin, ((0, 0), (0, CP - C)))
    b_cat = jnp.pad(b_lin, (0, CP - C)).reshape(1, CP)
    out = pl.pallas_call(
        _fin_kernel,
        out_shape=jax.ShapeDtypeStruct((G, CP), jnp.float32),
    )(pooled, w_cat, b_cat)
    return out[:, :C]


# speedup vs baseline: 5.5302x; 1.0104x over previous
import functools

import jax
import jax.numpy as jnp
from jax import lax
from jax.experimental import pallas as pl
from jax.experimental.pallas import tpu as pltpu
from jax.experimental.pallas import tpu_sc as plsc

N = 10000
E = 160000
D = 256
H = 512
C = 10
G = 128

NC = 2
NS = 16
L = 16

NPAD = 10240
EPT = E // NS
EPTP = 10240
CH = 80
ECH = EPTP // CH
NQ = 4
CPQ = ECH // NQ
DH = D // 2
GP = 144
NPA = 10112

_mesh = plsc.VectorSubcoreMesh(core_axis_name="c", subcore_axis_name="s")


def _agg_kernel(x2, srcp, dstp, z2d, z1d, agg2, degp,
                src_v, dst_v, rows0, rows1, ones_v,
                gsem0, gsem1, ssem0, ssem1, dsem0, dsem1,
                agg_sh, deg_sh):
    c = lax.axis_index("c")
    s = lax.axis_index("s")
    ones16 = jnp.full((L,), 1.0, jnp.float32)

    pltpu.sync_copy(z2d.at[pl.ds(0, 632)], agg_sh.at[pl.ds(s * 632, 632)])
    pltpu.sync_copy(z1d.at[pl.ds(s * 640, 640)],
                    deg_sh.at[pl.ds(s * 640, 640)])
    for t in range(CH // L):
        ones_v[pl.ds(t * L, L)] = ones16

    plsc.subcore_barrier()

    rows = (rows0, rows1)
    gsems = (gsem0, gsem1)
    ssems = (ssem0, ssem1)
    dsems = (dsem0, dsem1)

    def gather_start(k, b):
        pltpu.async_copy(x2.at[c].at[src_v.at[k]], rows[b], gsems[b])

    def gather_wait(b):
        pltpu.make_async_copy(x2.at[c].at[src_v.at[0]], rows[b],
                              gsems[b]).wait()

    def scatter_start(k, b):
        pltpu.async_copy(rows[b], agg_sh.at[dst_v.at[k]], ssems[b], add=True)

    def scatter_wait(b):
        pltpu.make_async_copy(rows[b], agg_sh.at[dst_v.at[0]],
                              ssems[b]).wait()

    def degsc_start(k, b):
        pltpu.async_copy(ones_v, deg_sh.at[dst_v.at[k]], dsems[b], add=True)

    def degsc_wait(b):
        pltpu.make_async_copy(ones_v, deg_sh.at[dst_v.at[0]],
                              dsems[b]).wait()

    for q in range(NQ):
        pltpu.sync_copy(srcp.at[s].at[q], src_v)
        pltpu.sync_copy(dstp.at[s].at[q], dst_v)
        deg_core = 0 if q < NQ // 2 else 1
        pltpu.async_copy(x2.at[c].at[src_v.at[0]], rows0, gsem0)

        def super_chunk(g, carry):
            for b in range(2):
                k = g * 2 + b

                @pl.when(k >= 1)
                def _():
                    scatter_wait(1 - b)

                @pl.when(k + 1 < CPQ)
                def _():
                    gather_start(k + 1, 1 - b)

                gather_wait(b)
                scatter_start(k, b)

                @pl.when(c == deg_core)
                def _():
                    @pl.when(k >= 1)
                    def _():
                        degsc_wait(1 - b)
                    degsc_start(k, b)

            return carry

        lax.fori_loop(0, CPQ // 2, super_chunk, 0, unroll=False)

        scatter_wait(1)

        @pl.when(c == deg_core)
        def _():
            degsc_wait(1)

    plsc.subcore_barrier()

    pltpu.sync_copy(agg_sh.at[pl.ds(s * 632, 632)],
                    agg2.at[c].at[pl.ds(s * 632, 632)])
    pltpu.sync_copy(deg_sh.at[pl.ds(s * 640, 640)],
                    degp.at[c].at[pl.ds(s * 640, 640)])


_agg_call = pl.kernel(
    _agg_kernel,
    out_type=[
        jax.ShapeDtypeStruct((NC, NPAD, DH), jnp.float32),
        jax.ShapeDtypeStruct((NC, NPAD), jnp.float32),
    ],
    mesh=_mesh,
    scratch_types=[
        pltpu.VMEM((CPQ, CH), jnp.int32),
        pltpu.VMEM((CPQ, CH), jnp.int32),
        pltpu.VMEM((CH, DH), jnp.float32),
        pltpu.VMEM((CH, DH), jnp.float32),
        pltpu.VMEM((CH,), jnp.float32),
        pltpu.SemaphoreType.DMA,
        pltpu.SemaphoreType.DMA,
        pltpu.SemaphoreType.DMA,
        pltpu.SemaphoreType.DMA,
        pltpu.SemaphoreType.DMA,
        pltpu.SemaphoreType.DMA,
        pltpu.VMEM_SHARED((NPA, DH), jnp.float32),
        pltpu.VMEM_SHARED((NPAD,), jnp.float32),
    ],
    compiler_params=pltpu.CompilerParams(needs_layout_passes=False),
)


def _mm_kernel(agg_ref, x_ref, deg_ref, wl_ref, wr_ref, bl_ref, out_ref):
    deg = deg_ref[0] + deg_ref[1]
    inv = 1.0 / jnp.maximum(deg, 1.0)
    ml = (agg_ref[0] * inv).astype(jnp.bfloat16)
    mh = (agg_ref[1] * inv).astype(jnp.bfloat16)
    acc = jnp.dot(ml, wl_ref[:DH], preferred_element_type=jnp.float32)
    acc += jnp.dot(mh, wl_ref[DH:], preferred_element_type=jnp.float32)
    acc += jnp.dot(x_ref[...], wr_ref[...], preferred_element_type=jnp.float32)
    acc += bl_ref[...]
    out_ref[...] = acc * 0.5 * (1.0 + lax.erf(acc * (2.0 ** -0.5)))


def _pool_kernel(h, batchp, pooled,
                 b_v, cnt_loc, cnt_dma, cntall_v, hbuf, acc_ts, pool_buf,
                 cntall_sh, pooled_sh):
    c = lax.axis_index("c")
    s = lax.axis_index("s")
    w = c * NS + s
    ones16 = jnp.full((L,), 1.0, jnp.float32)
    iota16 = lax.broadcasted_iota(jnp.int32, (L,), 0)

    pltpu.sync_copy(batchp.at[s], b_v)
    for t in range(GP // L):
        cnt_loc[pl.ds(t * L, L)] = jnp.zeros((L,), jnp.float32)
    for j in range(640 // L):
        idx16 = b_v[pl.ds(j * L, L)]
        plsc.addupdate_scatter(cnt_loc, [idx16], ones16)
    for t in range(GP // L):
        cnt_dma[pl.ds(t * L, L)] = cnt_loc[pl.ds(t * L, L)]
    pltpu.sync_copy(cnt_dma, cntall_sh.at[s])
    plsc.subcore_barrier()
    pltpu.sync_copy(cntall_sh, cntall_v)

    cnt_chunks = []
    for t in range(GP // L):
        acc = cntall_v[0, pl.ds(t * L, L)]
        for p in range(1, NS):
            acc = acc + cntall_v[p, pl.ds(t * L, L)]
        cnt_chunks.append(acc)

    def graph_scalars(g):
        start = jnp.float32(0)
        cnt = jnp.float32(0)
        for t, ch in enumerate(cnt_chunks):
            ids = t * L + iota16
            start = start + jnp.sum(jnp.where(ids < g, ch, 0.0), axis=0)
            cnt = cnt + jnp.sum(jnp.where(ids == g, ch, 0.0), axis=0)
        return start.astype(jnp.int32), cnt.astype(jnp.int32), cnt

    zero16 = jnp.zeros((L,), jnp.float32)
    pinf16 = jnp.full((L,), jnp.inf, jnp.float32)
    ninf16 = jnp.full((L,), -jnp.inf, jnp.float32)

    for i in range(4):
        g = w * 4 + i
        start, cnt, cnt_f = graph_scalars(g)

        for t in range(H // L):
            acc_ts[0, pl.ds(t * L, L)] = zero16
            acc_ts[1, pl.ds(t * L, L)] = pinf16
            acc_ts[2, pl.ds(t * L, L)] = ninf16

        def body(state):
            p = state
            base = start + p
            a = (base // 8) * 8
            off = base - a
            pltpu.sync_copy(h.at[pl.ds(a, 136)], hbuf)
            rows_here = jnp.minimum(128, cnt - p)
            for cg in range(4):
                accs = []
                for k in range(8):
                    col = cg * 128 + k * L
                    accs.append(acc_ts[0, pl.ds(col, L)])
                    accs.append(acc_ts[1, pl.ds(col, L)])
                    accs.append(acc_ts[2, pl.ds(col, L)])

                def row_body(r, accs):
                    accs = list(accs)
                    for k in range(8):
                        col = cg * 128 + k * L
                        v = hbuf[off + r, pl.ds(col, L)]
                        accs[3 * k] = accs[3 * k] + v
                        accs[3 * k + 1] = jnp.minimum(accs[3 * k + 1], v)
                        accs[3 * k + 2] = jnp.maximum(accs[3 * k + 2], v)
                    return tuple(accs)

                accs = lax.fori_loop(0, rows_here, row_body, tuple(accs),
                                     unroll=False)
                for k in range(8):
                    col = cg * 128 + k * L
                    acc_ts[0, pl.ds(col, L)] = accs[3 * k]
                    acc_ts[1, pl.ds(col, L)] = accs[3 * k + 1]
                    acc_ts[2, pl.ds(col, L)] = accs[3 * k + 2]
            return p + 128

        lax.while_loop(lambda p: p < cnt, body, jnp.int32(0))

        cnt16 = jnp.zeros((L,), jnp.float32) + jnp.maximum(cnt_f, 1.0)
        for t in range(H // L):
            pool_buf[i, pl.ds(t * L, L)] = acc_ts[0, pl.ds(t * L, L)] / cnt16
            pool_buf[i, pl.ds(H + t * L, L)] = acc_ts[1, pl.ds(t * L, L)]
            pool_buf[i, pl.ds(2 * H + t * L, L)] = acc_ts[2, pl.ds(t * L, L)]

    pltpu.sync_copy(pool_buf, pooled_sh.at[pl.ds(s * 4, 4)])
    plsc.subcore_barrier()

    @pl.when(s < 8)
    def _():
        pltpu.sync_copy(pooled_sh.at[pl.ds(s * 8, 8)],
                        pooled.at[pl.ds(c * 64 + s * 8, 8)])


_pool_call = pl.kernel(
    _pool_kernel,
    out_type=[jax.ShapeDtypeStruct((G, 3 * H), jnp.float32)],
    mesh=_mesh,
    scratch_types=[
        pltpu.VMEM((640,), jnp.int32),
        pltpu.VMEM((GP,), jnp.float32),
        pltpu.VMEM((GP,), jnp.float32),
        pltpu.VMEM((NS, GP), jnp.float32),
        pltpu.VMEM((136, H), jnp.float32),
        pltpu.VMEM((3, H), jnp.float32),
        pltpu.VMEM((4, 3 * H), jnp.float32),
        pltpu.VMEM_SHARED((NS, GP), jnp.float32),
        pltpu.VMEM_SHARED((64, 3 * H), jnp.float32),
    ],
    compiler_params=pltpu.CompilerParams(needs_layout_passes=False),
)


def _fin_kernel(p_ref, w_ref, b_ref, out_ref):
    out_ref[...] = jnp.dot(p_ref[...], w_ref[...],
                           preferred_element_type=jnp.float32) + b_ref[...]


def kernel(x, edge_index, batch, W_l, b_l, W_r, W_lin, b_lin):
    src = edge_index[0]
    dst = edge_index[1]

    x_pad = jnp.pad(x, ((0, NPAD - N), (0, 0)))
    x2 = x_pad.reshape(NPAD, 2, DH).transpose(1, 0, 2)
    srcp = jnp.pad(src.reshape(NS, EPT), ((0, 0), (0, EPTP - EPT))
                   ).reshape(NS, NQ, CPQ, CH)
    dstp = jnp.pad(dst.reshape(NS, EPT), ((0, 0), (0, EPTP - EPT)),
                   constant_values=NPA - 8).reshape(NS, NQ, CPQ, CH)
    z2d = jnp.zeros((640, DH), jnp.float32)
    z1d = jnp.zeros((NPAD,), jnp.float32)

    agg2, degp = _agg_call(x2, srcp, dstp, z2d, z1d)

    degp2 = degp.reshape(NC, NPAD, 1)
    bl2 = b_l.reshape(1, H)
    x_bf = x_pad.astype(jnp.bfloat16)
    wl_bf = W_l.astype(jnp.bfloat16)
    wr_bf = W_r.astype(jnp.bfloat16)
    h = pl.pallas_call(
        _mm_kernel,
        grid=(NPAD // 256,),
        in_specs=[
            pl.BlockSpec((NC, 256, DH), lambda i: (0, i, 0)),
            pl.BlockSpec((256, D), lambda i: (i, 0)),
            pl.BlockSpec((NC, 256, 1), lambda i: (0, i, 0)),
            pl.BlockSpec((D, H), lambda i: (0, 0)),
            pl.BlockSpec((D, H), lambda i: (0, 0)),
            pl.BlockSpec((1, H), lambda i: (0, 0)),
        ],
        out_specs=pl.BlockSpec((256, H), lambda i: (i, 0)),
        out_shape=jax.ShapeDtypeStruct((NPAD, H), jnp.float32),
    )(agg2, x_bf, degp2, wl_bf, wr_bf, bl2)

    batchp = jnp.pad(batch, (0, NPAD - N), constant_values=G).reshape(NS, 640)
    (pooled,) = _pool_call(h, batchp)

    CP = 128
    w_cat = jnp.pad(W_lin, ((0, 0), (0, CP - C)))
    b_cat = jnp.pad(b_lin, (0, CP - C)).reshape(1, CP)
    out = pl.pallas_call(
        _fin_kernel,
        out_shape=jax.ShapeDtypeStruct((G, CP), jnp.float32),
    )(pooled, w_cat, b_cat)
    return out[:, :C]
